# baseline (speedup 1.0000x reference)
import jax
import jax.numpy as jnp
import numpy as np
from jax.experimental import pallas as pl


def _gather(x, idx):
    return jax.vmap(lambda xb, ib: xb[ib])(x, idx)


def _sqdist(a, b):
    aa = jnp.sum(a * a, axis=-1)[:, :, None]
    bb = jnp.sum(b * b, axis=-1)[:, None, :]
    ab = jnp.einsum('bnc,bmc->bnm', a, b)
    return jnp.maximum(aa + bb - 2.0 * ab, 0.0)


def _knn(k, support, query):
    d = _sqdist(query, support)
    negd, idx = jax.lax.top_k(-d, k)
    return idx, -negd


def _ball_query(radius, k, support, query):
    idx, d = _knn(k, support, query)
    mask = d <= radius * radius
    return jnp.where(mask, idx, idx[:, :, :1])


def _bn(x, g, b, eps=1e-5):
    axes = tuple(range(x.ndim - 1))
    m = jnp.mean(x, axis=axes, keepdims=True)
    v = jnp.var(x, axis=axes, keepdims=True)
    return (x - m) / jnp.sqrt(v + eps) * g + b


def _ln(x, g, b, eps=1e-5):
    m = jnp.mean(x, axis=-1, keepdims=True)
    v = jnp.var(x, axis=-1, keepdims=True)
    return (x - m) / jnp.sqrt(v + eps) * g + b


def _ff(xyz, feat, idx, Wg, bg, Wf, bf):
    nx = _gather(xyz, idx)
    rel = nx - xyz[:, :, None, :]
    geo = jax.nn.relu(rel @ Wg + bg)
    nf = _gather(feat, idx)
    h = jax.nn.relu(jnp.concatenate([nf, geo], axis=-1) @ Wf + bf)
    return jnp.max(h, axis=2)


def _agg(xyz1, xyz2, f1, f2, idx, Wq, Wk, Wv, Wgeo, Wo, bo, H):
    B, N2, k = idx.shape
    C = Wq.shape[1]
    dh = C // H
    nf = _gather(f1, idx)
    nx = _gather(xyz1, idx)
    rel = nx - xyz2[:, :, None, :]
    kv = nf + rel @ Wgeo
    q = (f2 @ Wq).reshape(B, N2, H, dh)
    kk = (kv @ Wk).reshape(B, N2, k, H, dh)
    v = (kv @ Wv).reshape(B, N2, k, H, dh)
    att = jax.nn.softmax(jnp.einsum('bnhd,bnkhd->bnhk', q, kk) / np.sqrt(dh), axis=-1)
    o = jnp.einsum('bnhk,bnkhd->bnhd', att, v).reshape(B, N2, C)
    return jax.nn.relu(o @ Wo + bo)


def _glob(f, p):
    B, N, C = f.shape
    H = 8
    dh = C // H
    h = _ln(f, p['ln1_g'], p['ln1_b'])
    q = (h @ p['Wqg']).reshape(B, N, H, dh)
    k = (h @ p['Wkg']).reshape(B, N, H, dh)
    v = (h @ p['Wvg']).reshape(B, N, H, dh)
    att = jax.nn.softmax(jnp.einsum('bnhd,bmhd->bhnm', q, k) / np.sqrt(dh), axis=-1)
    o = jnp.einsum('bhnm,bmhd->bnhd', att, v).reshape(B, N, C) @ p['Wog']
    f = f + o
    h = _ln(f, p['ln2_g'], p['ln2_b'])
    f = f + jax.nn.relu(h @ p['Wff1'] + p['bff1']) @ p['Wff2'] + p['bff2']
    return f


def _up(fc, fskip, xyz_c, xyz_f, W1, b1, W2, b2):
    d = _sqdist(xyz_f, xyz_c)
    negd, idx3 = jax.lax.top_k(-d, 3)
    d3 = jnp.maximum(-negd, 1e-10)
    w = 1.0 / d3
    w = w / jnp.sum(w, axis=-1, keepdims=True)
    f3 = _gather(fc, idx3)
    fi = jnp.sum(w[..., None] * f3, axis=2)
    h = jnp.concatenate([fi, fskip], axis=-1)
    h = jax.nn.relu(h @ W1 + b1)
    return jax.nn.relu(h @ W2 + b2)


def _head_body(feat_ref, We1_ref, be1_ref, g1_ref, b1_ref,
               We2_ref, be2_ref, g2_ref, b2_ref,
               We3_ref, be3_ref, g3_ref, b3_ref, out_ref):
    feat = feat_ref[...]

    def bn(x, g, b, eps=1e-5):
        m = jnp.mean(x, axis=(0, 1), keepdims=True)
        v = jnp.mean((x - m) * (x - m), axis=(0, 1), keepdims=True)
        return (x - m) / jnp.sqrt(v + eps) * g + b

    h = jax.nn.relu(bn(feat @ We1_ref[...] + be1_ref[...], g1_ref[...], b1_ref[...]))
    h = jax.nn.relu(bn(h @ We2_ref[...] + be2_ref[...], g2_ref[...], b2_ref[...]))
    h = bn(h @ We3_ref[...] + be3_ref[...], g3_ref[...], b3_ref[...])
    out_ref[...] = jax.nn.log_softmax(h, axis=-1)


def _head(feat, p):
    B, N, _ = feat.shape
    args = (feat, p['We1'], p['be1'], p['g1'], p['b1'],
            p['We2'], p['be2'], p['g2'], p['b2'],
            p['We3'], p['be3'], p['g3'], p['b3'])
    return pl.pallas_call(
        _head_body,
        out_shape=jax.ShapeDtypeStruct((B, N, 13), jnp.float32),
    )(*args)


def kernel(points, params):
    p = params
    xyz = points[..., 0:3]
    xyz2 = xyz[:, ::4]
    xyz3 = xyz2[:, ::4]
    idxs = {
        's0': _ball_query(0.1, 32, xyz, xyz),
        'a0': _knn(16, xyz, xyz2)[0],
        's1': _ball_query(0.2, 32, xyz2, xyz2),
        'a1': _knn(16, xyz2, xyz3)[0],
    }
    feat = jax.nn.relu(_bn(xyz @ p['W_emb'] + p['b_emb'], p['g_emb'], p['be_emb']))
    f1 = _ff(xyz, feat, idxs['s0'], p['Wg0'], p['bg0'], p['Wf0'], p['bf0'])
    enc0 = f1
    f1s = f1[:, ::4]
    feat = _agg(xyz, xyz2, f1, f1s, idxs['a0'], p['Wq0'], p['Wk0'], p['Wv0'], p['Wgeo0'], p['Wo0'], p['bo0'], 4)
    enc1 = feat
    f1b = _ff(xyz2, feat, idxs['s1'], p['Wg1'], p['bg1'], p['Wf1'], p['bf1'])
    f1bs = f1b[:, ::4]
    feat = _agg(xyz2, xyz3, f1b, f1bs, idxs['a1'], p['Wq1'], p['Wk1'], p['Wv1'], p['Wgeo1'], p['Wo1'], p['bo1'], 4)
    feat = _glob(feat, p)
    feat = _up(feat, enc1, xyz3, xyz2, p['Wu0a'], p['bu0a'], p['Wu0b'], p['bu0b'])
    feat = _up(feat, enc0, xyz2, xyz, p['Wu1a'], p['bu1a'], p['Wu1b'], p['bu1b'])
    return _head(feat, p)


# speedup vs baseline: 1.9096x; 1.9096x over previous
import functools

import jax
import jax.numpy as jnp
import numpy as np
from jax import lax
from jax.experimental import pallas as pl
from jax.experimental.pallas import tpu as pltpu
from jax.experimental.pallas import tpu_sc as plsc

_NC = 2
_NS = 16
_NW = _NC * _NS


@functools.cache
def _make_sc_gather(V, D, B):
    assert D % 16 == 0 and B % (8 * _NW) == 0
    b_per_w = B // _NW
    CH = min(128, b_per_w)
    n_ch = b_per_w // CH
    assert b_per_w % CH == 0
    mesh = plsc.VectorSubcoreMesh(core_axis_name="c", subcore_axis_name="s")

    @functools.partial(
        pl.kernel, mesh=mesh,
        out_type=jax.ShapeDtypeStruct((B, D), jnp.float32),
        scratch_types=[
            pltpu.VMEM((CH,), jnp.int32),
            pltpu.VMEM((CH, D), jnp.float32),
            pltpu.SemaphoreType.DMA,
        ],
    )
    def k(table_hbm, idx_hbm, out_hbm, idx_v, rows_v, sem):
        wid = lax.axis_index("s") * _NC + lax.axis_index("c")
        base = wid * b_per_w

        def chunk(i, carry):
            off = base + i * CH
            pltpu.sync_copy(idx_hbm.at[pl.ds(off, CH)], idx_v)
            pltpu.async_copy(table_hbm.at[idx_v], rows_v, sem).wait()
            pltpu.sync_copy(rows_v, out_hbm.at[pl.ds(off, CH)])
            return carry

        lax.fori_loop(0, n_ch, chunk, 0)

    return k


def _sc_gather(table, idx):
    Bb, V, D = table.shape
    _, N, k = idx.shape
    Dp = ((D + 127) // 128) * 128
    if Dp != D:
        table = jnp.pad(table, ((0, 0), (0, 0), (0, Dp - D)))
    off = (jnp.arange(Bb, dtype=jnp.int32) * V)[:, None, None]
    flat_idx = (idx.astype(jnp.int32) + off).reshape(-1)
    out = _make_sc_gather(Bb * V, Dp, flat_idx.shape[0])(table.reshape(-1, Dp), flat_idx)
    return out.reshape(Bb, N, k, Dp)[..., :D]


def _pad16(xyz):
    return jnp.pad(xyz, ((0, 0), (0, 0), (0, 13)))


def _sqdist(a, b):
    aa = jnp.sum(a * a, axis=-1)[:, :, None]
    bb = jnp.sum(b * b, axis=-1)[:, None, :]
    ab = jnp.einsum('bnc,bmc->bnm', a, b)
    return jnp.maximum(aa + bb - 2.0 * ab, 0.0)


def _knn(k, support, query):
    d = _sqdist(query, support)
    negd, idx = jax.lax.top_k(-d, k)
    return idx, -negd


def _ball_query(radius, k, support, query):
    idx, d = _knn(k, support, query)
    mask = d <= radius * radius
    return jnp.where(mask, idx, idx[:, :, :1])


def _bn(x, g, b, eps=1e-5):
    axes = tuple(range(x.ndim - 1))
    m = jnp.mean(x, axis=axes, keepdims=True)
    v = jnp.var(x, axis=axes, keepdims=True)
    return (x - m) / jnp.sqrt(v + eps) * g + b


def _ln(x, g, b, eps=1e-5):
    m = jnp.mean(x, axis=-1, keepdims=True)
    v = jnp.var(x, axis=-1, keepdims=True)
    return (x - m) / jnp.sqrt(v + eps) * g + b


def _ff(xyz, feat, idx, Wg, bg, Wf, bf):
    g = _sc_gather(jnp.concatenate([_pad16(xyz), feat], axis=-1), idx)
    nx = g[..., :3]
    nf = g[..., 16:]
    rel = nx - xyz[:, :, None, :]
    geo = jax.nn.relu(rel @ Wg + bg)
    h = jax.nn.relu(jnp.concatenate([nf, geo], axis=-1) @ Wf + bf)
    return jnp.max(h, axis=2)


def _agg(xyz1, xyz2, f1, f2, idx, Wq, Wk, Wv, Wgeo, Wo, bo, H):
    B, N2, k = idx.shape
    C = Wq.shape[1]
    dh = C // H
    g = _sc_gather(jnp.concatenate([_pad16(xyz1), f1], axis=-1), idx)
    nx = g[..., :3]
    nf = g[..., 16:]
    rel = nx - xyz2[:, :, None, :]
    kv = nf + rel @ Wgeo
    q = (f2 @ Wq).reshape(B, N2, H, dh)
    kk = (kv @ Wk).reshape(B, N2, k, H, dh)
    v = (kv @ Wv).reshape(B, N2, k, H, dh)
    att = jax.nn.softmax(jnp.einsum('bnhd,bnkhd->bnhk', q, kk) / np.sqrt(dh), axis=-1)
    o = jnp.einsum('bnhk,bnkhd->bnhd', att, v).reshape(B, N2, C)
    return jax.nn.relu(o @ Wo + bo)


def _glob(f, p):
    B, N, C = f.shape
    H = 8
    dh = C // H
    h = _ln(f, p['ln1_g'], p['ln1_b'])
    q = (h @ p['Wqg']).reshape(B, N, H, dh)
    k = (h @ p['Wkg']).reshape(B, N, H, dh)
    v = (h @ p['Wvg']).reshape(B, N, H, dh)
    att = jax.nn.softmax(jnp.einsum('bnhd,bmhd->bhnm', q, k) / np.sqrt(dh), axis=-1)
    o = jnp.einsum('bhnm,bmhd->bnhd', att, v).reshape(B, N, C) @ p['Wog']
    f = f + o
    h = _ln(f, p['ln2_g'], p['ln2_b'])
    f = f + jax.nn.relu(h @ p['Wff1'] + p['bff1']) @ p['Wff2'] + p['bff2']
    return f


def _up(fc, fskip, xyz_c, xyz_f, W1, b1, W2, b2):
    d = _sqdist(xyz_f, xyz_c)
    negd, idx3 = jax.lax.top_k(-d, 3)
    d3 = jnp.maximum(-negd, 1e-10)
    w = 1.0 / d3
    w = w / jnp.sum(w, axis=-1, keepdims=True)
    f3 = _sc_gather(fc, idx3)
    fi = jnp.sum(w[..., None] * f3, axis=2)
    h = jnp.concatenate([fi, fskip], axis=-1)
    h = jax.nn.relu(h @ W1 + b1)
    return jax.nn.relu(h @ W2 + b2)


def _head_body(feat_ref, We1_ref, be1_ref, g1_ref, b1_ref,
               We2_ref, be2_ref, g2_ref, b2_ref,
               We3_ref, be3_ref, g3_ref, b3_ref, out_ref):
    feat = feat_ref[...]

    def bn(x, g, b, eps=1e-5):
        m = jnp.mean(x, axis=(0, 1), keepdims=True)
        v = jnp.mean((x - m) * (x - m), axis=(0, 1), keepdims=True)
        return (x - m) / jnp.sqrt(v + eps) * g + b

    h = jax.nn.relu(bn(feat @ We1_ref[...] + be1_ref[...], g1_ref[...], b1_ref[...]))
    h = jax.nn.relu(bn(h @ We2_ref[...] + be2_ref[...], g2_ref[...], b2_ref[...]))
    h = bn(h @ We3_ref[...] + be3_ref[...], g3_ref[...], b3_ref[...])
    out_ref[...] = jax.nn.log_softmax(h, axis=-1)


def _head(feat, p):
    B, N, _ = feat.shape
    args = (feat, p['We1'], p['be1'], p['g1'], p['b1'],
            p['We2'], p['be2'], p['g2'], p['b2'],
            p['We3'], p['be3'], p['g3'], p['b3'])
    return pl.pallas_call(
        _head_body,
        out_shape=jax.ShapeDtypeStruct((B, N, 13), jnp.float32),
    )(*args)


def kernel(points, params):
    p = params
    xyz = points[..., 0:3]
    xyz2 = xyz[:, ::4]
    xyz3 = xyz2[:, ::4]
    idxs = {
        's0': _ball_query(0.1, 32, xyz, xyz),
        'a0': _knn(16, xyz, xyz2)[0],
        's1': _ball_query(0.2, 32, xyz2, xyz2),
        'a1': _knn(16, xyz2, xyz3)[0],
    }
    feat = jax.nn.relu(_bn(xyz @ p['W_emb'] + p['b_emb'], p['g_emb'], p['be_emb']))
    f1 = _ff(xyz, feat, idxs['s0'], p['Wg0'], p['bg0'], p['Wf0'], p['bf0'])
    enc0 = f1
    f1s = f1[:, ::4]
    feat = _agg(xyz, xyz2, f1, f1s, idxs['a0'], p['Wq0'], p['Wk0'], p['Wv0'], p['Wgeo0'], p['Wo0'], p['bo0'], 4)
    enc1 = feat
    f1b = _ff(xyz2, feat, idxs['s1'], p['Wg1'], p['bg1'], p['Wf1'], p['bf1'])
    f1bs = f1b[:, ::4]
    feat = _agg(xyz2, xyz3, f1b, f1bs, idxs['a1'], p['Wq1'], p['Wk1'], p['Wv1'], p['Wgeo1'], p['Wo1'], p['bo1'], 4)
    feat = _glob(feat, p)
    feat = _up(feat, enc1, xyz3, xyz2, p['Wu0a'], p['bu0a'], p['Wu0b'], p['bu0b'])
    feat = _up(feat, enc0, xyz2, xyz, p['Wu1a'], p['bu1a'], p['Wu1b'], p['bu1b'])
    return _head(feat, p)


# speedup vs baseline: 5.6549x; 2.9613x over previous
import functools

import jax
import jax.numpy as jnp
import numpy as np
from jax import lax
from jax.experimental import pallas as pl
from jax.experimental.pallas import tpu as pltpu
from jax.experimental.pallas import tpu_sc as plsc

_NC = 2
_NS = 16
_NW = _NC * _NS


@functools.cache
def _make_sc_gather(V, D, B):
    assert D % 16 == 0 and B % (8 * _NW) == 0
    b_per_w = B // _NW
    CH = min(128, b_per_w)
    n_ch = b_per_w // CH
    assert b_per_w % CH == 0
    mesh = plsc.VectorSubcoreMesh(core_axis_name="c", subcore_axis_name="s")

    @functools.partial(
        pl.kernel, mesh=mesh,
        out_type=jax.ShapeDtypeStruct((B, D), jnp.float32),
        scratch_types=[
            pltpu.VMEM((CH,), jnp.int32),
            pltpu.VMEM((CH, D), jnp.float32),
            pltpu.SemaphoreType.DMA,
        ],
    )
    def k(table_hbm, idx_hbm, out_hbm, idx_v, rows_v, sem):
        wid = lax.axis_index("s") * _NC + lax.axis_index("c")
        base = wid * b_per_w

        def chunk(i, carry):
            off = base + i * CH
            pltpu.sync_copy(idx_hbm.at[pl.ds(off, CH)], idx_v)
            pltpu.async_copy(table_hbm.at[idx_v], rows_v, sem).wait()
            pltpu.sync_copy(rows_v, out_hbm.at[pl.ds(off, CH)])
            return carry

        lax.fori_loop(0, n_ch, chunk, 0)

    return k


def _sc_gather(table, idx):
    Bb, V, D = table.shape
    _, N, k = idx.shape
    Dp = ((D + 127) // 128) * 128
    if Dp != D:
        table = jnp.pad(table, ((0, 0), (0, 0), (0, Dp - D)))
    off = (jnp.arange(Bb, dtype=jnp.int32) * V)[:, None, None]
    flat_idx = (idx.astype(jnp.int32) + off).reshape(-1)
    out = _make_sc_gather(Bb * V, Dp, flat_idx.shape[0])(table.reshape(-1, Dp), flat_idx)
    return out.reshape(Bb, N, k, Dp)[..., :D]


def _pad16(xyz):
    return jnp.pad(xyz, ((0, 0), (0, 0), (0, 13)))


def _make_knn_tc(B, Nq, Ns, K, QB, r2, interpret=False):
    grid = (B, Nq // QB)

    def body(q_ref, s_ref, bb_ref, oidx_ref, d_scr):
        q = q_ref[0]
        s = s_ref[0]
        bb = bb_ref[0]
        ab = jax.lax.dot_general(q, s, dimension_numbers=(((1,), (1,)), ((), ())),
                                 preferred_element_type=jnp.float32)
        aa = jnp.sum(q * q, axis=1, keepdims=True)
        d_scr[...] = jnp.maximum(aa + bb - 2.0 * ab, 0.0)
        iota = jax.lax.broadcasted_iota(jnp.int32, (QB, Ns), 1)
        kio = jax.lax.broadcasted_iota(jnp.int32, (QB, K), 1)

        def step(j, carry):
            am0, acc = carry
            d = d_scr[...]
            m = jnp.min(d, axis=1, keepdims=True)
            sel = jnp.where(d <= m, iota, Ns)
            am = jnp.min(sel, axis=1, keepdims=True)
            am0 = jnp.where(j == 0, am, am0)
            res = am if r2 is None else jnp.where(m <= r2, am, am0)
            acc = jnp.where(kio == j, res, acc)
            d_scr[...] = jnp.where(sel == am, jnp.float32(jnp.inf), d)
            return am0, acc

        _, acc = jax.lax.fori_loop(
            0, K, step,
            (jnp.zeros((QB, 1), jnp.int32), jnp.zeros((QB, K), jnp.int32)))
        oidx_ref[0] = acc

    return pl.pallas_call(
        body,
        grid=grid,
        in_specs=[
            pl.BlockSpec((1, QB, 3), lambda b, i: (b, i, 0)),
            pl.BlockSpec((1, Ns, 3), lambda b, i: (b, 0, 0)),
            pl.BlockSpec((1, 1, Ns), lambda b, i: (b, 0, 0)),
        ],
        out_specs=pl.BlockSpec((1, QB, K), lambda b, i: (b, i, 0)),
        out_shape=jax.ShapeDtypeStruct((B, Nq, K), jnp.int32),
        scratch_shapes=[pltpu.VMEM((QB, Ns), jnp.float32)],
        interpret=interpret,
    )


def _knn_tc(k, support, query, r2=None, qb=128, interpret=False):
    B, Nq, _ = query.shape
    Ns = support.shape[1]
    bb = jnp.sum(support * support, axis=-1)[:, None, :]
    fn = _make_knn_tc(B, Nq, Ns, k, qb, r2, interpret)
    return fn(query, support, bb)


def _sqdist(a, b):
    aa = jnp.sum(a * a, axis=-1)[:, :, None]
    bb = jnp.sum(b * b, axis=-1)[:, None, :]
    ab = jnp.einsum('bnc,bmc->bnm', a, b)
    return jnp.maximum(aa + bb - 2.0 * ab, 0.0)


def _knn(k, support, query):
    d = _sqdist(query, support)
    negd, idx = jax.lax.top_k(-d, k)
    return idx, -negd


def _ball_query(radius, k, support, query):
    idx, d = _knn(k, support, query)
    mask = d <= radius * radius
    return jnp.where(mask, idx, idx[:, :, :1])


def _bn(x, g, b, eps=1e-5):
    axes = tuple(range(x.ndim - 1))
    m = jnp.mean(x, axis=axes, keepdims=True)
    v = jnp.var(x, axis=axes, keepdims=True)
    return (x - m) / jnp.sqrt(v + eps) * g + b


def _ln(x, g, b, eps=1e-5):
    m = jnp.mean(x, axis=-1, keepdims=True)
    v = jnp.var(x, axis=-1, keepdims=True)
    return (x - m) / jnp.sqrt(v + eps) * g + b


def _ff(xyz, feat, idx, Wg, bg, Wf, bf):
    g = _sc_gather(jnp.concatenate([_pad16(xyz), feat], axis=-1), idx)
    nx = g[..., :3]
    nf = g[..., 16:]
    rel = nx - xyz[:, :, None, :]
    geo = jax.nn.relu(rel @ Wg + bg)
    h = jax.nn.relu(jnp.concatenate([nf, geo], axis=-1) @ Wf + bf)
    return jnp.max(h, axis=2)


def _agg(xyz1, xyz2, f1, f2, idx, Wq, Wk, Wv, Wgeo, Wo, bo, H):
    B, N2, k = idx.shape
    C = Wq.shape[1]
    dh = C // H
    g = _sc_gather(jnp.concatenate([_pad16(xyz1), f1], axis=-1), idx)
    nx = g[..., :3]
    nf = g[..., 16:]
    rel = nx - xyz2[:, :, None, :]
    kv = nf + rel @ Wgeo
    q = (f2 @ Wq).reshape(B, N2, H, dh)
    kk = (kv @ Wk).reshape(B, N2, k, H, dh)
    v = (kv @ Wv).reshape(B, N2, k, H, dh)
    att = jax.nn.softmax(jnp.einsum('bnhd,bnkhd->bnhk', q, kk) / np.sqrt(dh), axis=-1)
    o = jnp.einsum('bnhk,bnkhd->bnhd', att, v).reshape(B, N2, C)
    return jax.nn.relu(o @ Wo + bo)


def _glob(f, p):
    B, N, C = f.shape
    H = 8
    dh = C // H
    h = _ln(f, p['ln1_g'], p['ln1_b'])
    q = (h @ p['Wqg']).reshape(B, N, H, dh)
    k = (h @ p['Wkg']).reshape(B, N, H, dh)
    v = (h @ p['Wvg']).reshape(B, N, H, dh)
    att = jax.nn.softmax(jnp.einsum('bnhd,bmhd->bhnm', q, k) / np.sqrt(dh), axis=-1)
    o = jnp.einsum('bhnm,bmhd->bnhd', att, v).reshape(B, N, C) @ p['Wog']
    f = f + o
    h = _ln(f, p['ln2_g'], p['ln2_b'])
    f = f + jax.nn.relu(h @ p['Wff1'] + p['bff1']) @ p['Wff2'] + p['bff2']
    return f


def _up(fc, fskip, xyz_c, xyz_f, W1, b1, W2, b2):
    d = _sqdist(xyz_f, xyz_c)
    negd, idx3 = jax.lax.top_k(-d, 3)
    d3 = jnp.maximum(-negd, 1e-10)
    w = 1.0 / d3
    w = w / jnp.sum(w, axis=-1, keepdims=True)
    f3 = _sc_gather(fc, idx3)
    fi = jnp.sum(w[..., None] * f3, axis=2)
    h = jnp.concatenate([fi, fskip], axis=-1)
    h = jax.nn.relu(h @ W1 + b1)
    return jax.nn.relu(h @ W2 + b2)


def _head_body(feat_ref, We1_ref, be1_ref, g1_ref, b1_ref,
               We2_ref, be2_ref, g2_ref, b2_ref,
               We3_ref, be3_ref, g3_ref, b3_ref, out_ref):
    feat = feat_ref[...]

    def bn(x, g, b, eps=1e-5):
        m = jnp.mean(x, axis=(0, 1), keepdims=True)
        v = jnp.mean((x - m) * (x - m), axis=(0, 1), keepdims=True)
        return (x - m) / jnp.sqrt(v + eps) * g + b

    h = jax.nn.relu(bn(feat @ We1_ref[...] + be1_ref[...], g1_ref[...], b1_ref[...]))
    h = jax.nn.relu(bn(h @ We2_ref[...] + be2_ref[...], g2_ref[...], b2_ref[...]))
    h = bn(h @ We3_ref[...] + be3_ref[...], g3_ref[...], b3_ref[...])
    out_ref[...] = jax.nn.log_softmax(h, axis=-1)


def _head(feat, p):
    B, N, _ = feat.shape
    args = (feat, p['We1'], p['be1'], p['g1'], p['b1'],
            p['We2'], p['be2'], p['g2'], p['b2'],
            p['We3'], p['be3'], p['g3'], p['b3'])
    return pl.pallas_call(
        _head_body,
        out_shape=jax.ShapeDtypeStruct((B, N, 13), jnp.float32),
    )(*args)


def kernel(points, params):
    p = params
    xyz = points[..., 0:3]
    xyz2 = xyz[:, ::4]
    xyz3 = xyz2[:, ::4]
    idxs = {
        's0': _knn_tc(32, xyz, xyz, r2=0.01),
        'a0': _knn_tc(16, xyz, xyz2),
        's1': _knn_tc(32, xyz2, xyz2, r2=0.04),
        'a1': _knn_tc(16, xyz2, xyz3),
    }
    feat = jax.nn.relu(_bn(xyz @ p['W_emb'] + p['b_emb'], p['g_emb'], p['be_emb']))
    f1 = _ff(xyz, feat, idxs['s0'], p['Wg0'], p['bg0'], p['Wf0'], p['bf0'])
    enc0 = f1
    f1s = f1[:, ::4]
    feat = _agg(xyz, xyz2, f1, f1s, idxs['a0'], p['Wq0'], p['Wk0'], p['Wv0'], p['Wgeo0'], p['Wo0'], p['bo0'], 4)
    enc1 = feat
    f1b = _ff(xyz2, feat, idxs['s1'], p['Wg1'], p['bg1'], p['Wf1'], p['bf1'])
    f1bs = f1b[:, ::4]
    feat = _agg(xyz2, xyz3, f1b, f1bs, idxs['a1'], p['Wq1'], p['Wk1'], p['Wv1'], p['Wgeo1'], p['Wo1'], p['bo1'], 4)
    feat = _glob(feat, p)
    feat = _up(feat, enc1, xyz3, xyz2, p['Wu0a'], p['bu0a'], p['Wu0b'], p['bu0b'])
    feat = _up(feat, enc0, xyz2, xyz, p['Wu1a'], p['bu1a'], p['Wu1b'], p['bu1b'])
    return _head(feat, p)


# speedup vs baseline: 5.9066x; 1.0445x over previous
import functools

import jax
import jax.numpy as jnp
import numpy as np
from jax import lax
from jax.experimental import pallas as pl
from jax.experimental.pallas import tpu as pltpu
from jax.experimental.pallas import tpu_sc as plsc

_NC = 2
_NS = 16
_NW = _NC * _NS


@functools.cache
def _make_sc_gather(V, D, B):
    assert D % 16 == 0 and B % (8 * _NW) == 0
    b_per_w = B // _NW
    CH = min(128, b_per_w)
    n_ch = b_per_w // CH
    assert b_per_w % CH == 0
    mesh = plsc.VectorSubcoreMesh(core_axis_name="c", subcore_axis_name="s")

    @functools.partial(
        pl.kernel, mesh=mesh,
        out_type=jax.ShapeDtypeStruct((B, D), jnp.float32),
        scratch_types=[
            pltpu.VMEM((CH,), jnp.int32),
            pltpu.VMEM((CH, D), jnp.float32),
            pltpu.SemaphoreType.DMA,
        ],
    )
    def k(table_hbm, idx_hbm, out_hbm, idx_v, rows_v, sem):
        wid = lax.axis_index("s") * _NC + lax.axis_index("c")
        base = wid * b_per_w

        def chunk(i, carry):
            off = base + i * CH
            pltpu.sync_copy(idx_hbm.at[pl.ds(off, CH)], idx_v)
            pltpu.async_copy(table_hbm.at[idx_v], rows_v, sem).wait()
            pltpu.sync_copy(rows_v, out_hbm.at[pl.ds(off, CH)])
            return carry

        lax.fori_loop(0, n_ch, chunk, 0)

    return k


def _sc_gather(table, idx):
    Bb, V, D = table.shape
    _, N, k = idx.shape
    Dp = ((D + 127) // 128) * 128
    if Dp != D:
        table = jnp.pad(table, ((0, 0), (0, 0), (0, Dp - D)))
    off = (jnp.arange(Bb, dtype=jnp.int32) * V)[:, None, None]
    flat_idx = (idx.astype(jnp.int32) + off).reshape(-1)
    out = _make_sc_gather(Bb * V, Dp, flat_idx.shape[0])(table.reshape(-1, Dp), flat_idx)
    return out.reshape(Bb, N, k, Dp)[..., :D]


def _pad16(xyz):
    return jnp.pad(xyz, ((0, 0), (0, 0), (0, 13)))


def _make_knn_tc(B, Nq, Ns, K, QB, r2, interpret=False):
    grid = (B, Nq // QB)

    def body(q_ref, s_ref, bb_ref, oidx_ref, d_scr):
        q = q_ref[0]
        s = s_ref[0]
        bb = bb_ref[0]
        ab = jax.lax.dot_general(q, s, dimension_numbers=(((1,), (1,)), ((), ())),
                                 preferred_element_type=jnp.float32)
        aa = jnp.sum(q * q, axis=1, keepdims=True)
        d_scr[...] = jnp.maximum(aa + bb - 2.0 * ab, 0.0)
        iota = jax.lax.broadcasted_iota(jnp.int32, (QB, Ns), 1)
        kio = jax.lax.broadcasted_iota(jnp.int32, (QB, K), 1)

        def step(j, carry):
            am0, acc = carry
            d = d_scr[...]
            m = jnp.min(d, axis=1, keepdims=True)
            sel = jnp.where(d <= m, iota, Ns)
            am = jnp.min(sel, axis=1, keepdims=True)
            am0 = jnp.where(j == 0, am, am0)
            res = am if r2 is None else jnp.where(m <= r2, am, am0)
            acc = jnp.where(kio == j, res, acc)
            d_scr[...] = jnp.where(sel == am, jnp.float32(jnp.inf), d)
            return am0, acc

        _, acc = jax.lax.fori_loop(
            0, K, step,
            (jnp.zeros((QB, 1), jnp.int32), jnp.zeros((QB, K), jnp.int32)))
        oidx_ref[0] = acc

    return pl.pallas_call(
        body,
        grid=grid,
        in_specs=[
            pl.BlockSpec((1, QB, 3), lambda b, i: (b, i, 0)),
            pl.BlockSpec((1, Ns, 3), lambda b, i: (b, 0, 0)),
            pl.BlockSpec((1, 1, Ns), lambda b, i: (b, 0, 0)),
        ],
        out_specs=pl.BlockSpec((1, QB, K), lambda b, i: (b, i, 0)),
        out_shape=jax.ShapeDtypeStruct((B, Nq, K), jnp.int32),
        scratch_shapes=[pltpu.VMEM((QB, Ns), jnp.float32)],
        interpret=interpret,
    )


def _knn_tc(k, support, query, r2=None, qb=256, interpret=False):
    B, Nq, _ = query.shape
    Ns = support.shape[1]
    bb = jnp.sum(support * support, axis=-1)[:, None, :]
    fn = _make_knn_tc(B, Nq, Ns, k, qb, r2, interpret)
    return fn(query, support, bb)


def _sqdist(a, b):
    aa = jnp.sum(a * a, axis=-1)[:, :, None]
    bb = jnp.sum(b * b, axis=-1)[:, None, :]
    ab = jnp.einsum('bnc,bmc->bnm', a, b)
    return jnp.maximum(aa + bb - 2.0 * ab, 0.0)


def _knn(k, support, query):
    d = _sqdist(query, support)
    negd, idx = jax.lax.top_k(-d, k)
    return idx, -negd


def _ball_query(radius, k, support, query):
    idx, d = _knn(k, support, query)
    mask = d <= radius * radius
    return jnp.where(mask, idx, idx[:, :, :1])


def _bn(x, g, b, eps=1e-5):
    axes = tuple(range(x.ndim - 1))
    m = jnp.mean(x, axis=axes, keepdims=True)
    v = jnp.var(x, axis=axes, keepdims=True)
    return (x - m) / jnp.sqrt(v + eps) * g + b


def _ln(x, g, b, eps=1e-5):
    m = jnp.mean(x, axis=-1, keepdims=True)
    v = jnp.var(x, axis=-1, keepdims=True)
    return (x - m) / jnp.sqrt(v + eps) * g + b


def _ff(xyz, feat, idx, Wg, bg, Wf, bf):
    g = _sc_gather(jnp.concatenate([_pad16(xyz), feat], axis=-1), idx)
    nx = g[..., :3]
    nf = g[..., 16:]
    rel = nx - xyz[:, :, None, :]
    geo = jax.nn.relu(rel @ Wg + bg)
    h = jax.nn.relu(jnp.concatenate([nf, geo], axis=-1) @ Wf + bf)
    return jnp.max(h, axis=2)


def _agg(xyz1, xyz2, f1, f2, idx, Wq, Wk, Wv, Wgeo, Wo, bo, H):
    B, N2, k = idx.shape
    C = Wq.shape[1]
    dh = C // H
    g = _sc_gather(jnp.concatenate([_pad16(xyz1), f1], axis=-1), idx)
    nx = g[..., :3]
    nf = g[..., 16:]
    rel = nx - xyz2[:, :, None, :]
    kv = nf + rel @ Wgeo
    q = (f2 @ Wq).reshape(B, N2, H, dh)
    kk = (kv @ Wk).reshape(B, N2, k, H, dh)
    v = (kv @ Wv).reshape(B, N2, k, H, dh)
    att = jax.nn.softmax(jnp.einsum('bnhd,bnkhd->bnhk', q, kk) / np.sqrt(dh), axis=-1)
    o = jnp.einsum('bnhk,bnkhd->bnhd', att, v).reshape(B, N2, C)
    return jax.nn.relu(o @ Wo + bo)


def _glob(f, p):
    B, N, C = f.shape
    H = 8
    dh = C // H
    h = _ln(f, p['ln1_g'], p['ln1_b'])
    q = (h @ p['Wqg']).reshape(B, N, H, dh)
    k = (h @ p['Wkg']).reshape(B, N, H, dh)
    v = (h @ p['Wvg']).reshape(B, N, H, dh)
    att = jax.nn.softmax(jnp.einsum('bnhd,bmhd->bhnm', q, k) / np.sqrt(dh), axis=-1)
    o = jnp.einsum('bhnm,bmhd->bnhd', att, v).reshape(B, N, C) @ p['Wog']
    f = f + o
    h = _ln(f, p['ln2_g'], p['ln2_b'])
    f = f + jax.nn.relu(h @ p['Wff1'] + p['bff1']) @ p['Wff2'] + p['bff2']
    return f


def _up(fc, fskip, xyz_c, xyz_f, W1, b1, W2, b2):
    d = _sqdist(xyz_f, xyz_c)
    negd, idx3 = jax.lax.top_k(-d, 3)
    d3 = jnp.maximum(-negd, 1e-10)
    w = 1.0 / d3
    w = w / jnp.sum(w, axis=-1, keepdims=True)
    f3 = _sc_gather(fc, idx3)
    fi = jnp.sum(w[..., None] * f3, axis=2)
    h = jnp.concatenate([fi, fskip], axis=-1)
    h = jax.nn.relu(h @ W1 + b1)
    return jax.nn.relu(h @ W2 + b2)


def _head_body(feat_ref, We1_ref, be1_ref, g1_ref, b1_ref,
               We2_ref, be2_ref, g2_ref, b2_ref,
               We3_ref, be3_ref, g3_ref, b3_ref, out_ref):
    feat = feat_ref[...]

    def bn(x, g, b, eps=1e-5):
        m = jnp.mean(x, axis=(0, 1), keepdims=True)
        v = jnp.mean((x - m) * (x - m), axis=(0, 1), keepdims=True)
        return (x - m) / jnp.sqrt(v + eps) * g + b

    h = jax.nn.relu(bn(feat @ We1_ref[...] + be1_ref[...], g1_ref[...], b1_ref[...]))
    h = jax.nn.relu(bn(h @ We2_ref[...] + be2_ref[...], g2_ref[...], b2_ref[...]))
    h = bn(h @ We3_ref[...] + be3_ref[...], g3_ref[...], b3_ref[...])
    out_ref[...] = jax.nn.log_softmax(h, axis=-1)


def _head(feat, p):
    B, N, _ = feat.shape
    args = (feat, p['We1'], p['be1'], p['g1'], p['b1'],
            p['We2'], p['be2'], p['g2'], p['b2'],
            p['We3'], p['be3'], p['g3'], p['b3'])
    return pl.pallas_call(
        _head_body,
        out_shape=jax.ShapeDtypeStruct((B, N, 13), jnp.float32),
    )(*args)


def kernel(points, params):
    p = params
    xyz = points[..., 0:3]
    xyz2 = xyz[:, ::4]
    xyz3 = xyz2[:, ::4]
    idxs = {
        's0': _knn_tc(32, xyz, xyz, r2=0.01),
        'a0': _knn_tc(16, xyz, xyz2),
        's1': _knn_tc(32, xyz2, xyz2, r2=0.04),
        'a1': _knn_tc(16, xyz2, xyz3),
    }
    feat = jax.nn.relu(_bn(xyz @ p['W_emb'] + p['b_emb'], p['g_emb'], p['be_emb']))
    f1 = _ff(xyz, feat, idxs['s0'], p['Wg0'], p['bg0'], p['Wf0'], p['bf0'])
    enc0 = f1
    f1s = f1[:, ::4]
    feat = _agg(xyz, xyz2, f1, f1s, idxs['a0'], p['Wq0'], p['Wk0'], p['Wv0'], p['Wgeo0'], p['Wo0'], p['bo0'], 4)
    enc1 = feat
    f1b = _ff(xyz2, feat, idxs['s1'], p['Wg1'], p['bg1'], p['Wf1'], p['bf1'])
    f1bs = f1b[:, ::4]
    feat = _agg(xyz2, xyz3, f1b, f1bs, idxs['a1'], p['Wq1'], p['Wk1'], p['Wv1'], p['Wgeo1'], p['Wo1'], p['bo1'], 4)
    feat = _glob(feat, p)
    feat = _up(feat, enc1, xyz3, xyz2, p['Wu0a'], p['bu0a'], p['Wu0b'], p['bu0b'])
    feat = _up(feat, enc0, xyz2, xyz, p['Wu1a'], p['bu1a'], p['Wu1b'], p['bu1b'])
    return _head(feat, p)


# speedup vs baseline: 6.2255x; 1.0540x over previous
import functools

import jax
import jax.numpy as jnp
import numpy as np
from jax import lax
from jax.experimental import pallas as pl
from jax.experimental.pallas import tpu as pltpu
from jax.experimental.pallas import tpu_sc as plsc

_NC = 2
_NS = 16
_NW = _NC * _NS


@functools.cache
def _make_sc_gather(V, D, B):
    assert D % 16 == 0 and B % (8 * _NW) == 0
    b_per_w = B // _NW
    CH = min(128, b_per_w)
    n_ch = b_per_w // CH
    assert b_per_w % CH == 0
    mesh = plsc.VectorSubcoreMesh(core_axis_name="c", subcore_axis_name="s")

    @functools.partial(
        pl.kernel, mesh=mesh,
        out_type=jax.ShapeDtypeStruct((B, D), jnp.float32),
        scratch_types=[
            pltpu.VMEM((CH,), jnp.int32),
            pltpu.VMEM((CH, D), jnp.float32),
            pltpu.SemaphoreType.DMA,
        ],
    )
    def k(table_hbm, idx_hbm, out_hbm, idx_v, rows_v, sem):
        wid = lax.axis_index("s") * _NC + lax.axis_index("c")
        base = wid * b_per_w

        def chunk(i, carry):
            off = base + i * CH
            pltpu.sync_copy(idx_hbm.at[pl.ds(off, CH)], idx_v)
            pltpu.async_copy(table_hbm.at[idx_v], rows_v, sem).wait()
            pltpu.sync_copy(rows_v, out_hbm.at[pl.ds(off, CH)])
            return carry

        lax.fori_loop(0, n_ch, chunk, 0)

    return k


def _sc_gather(table, idx):
    Bb, V, D = table.shape
    _, N, k = idx.shape
    Dp = ((D + 127) // 128) * 128
    if Dp != D:
        table = jnp.pad(table, ((0, 0), (0, 0), (0, Dp - D)))
    off = (jnp.arange(Bb, dtype=jnp.int32) * V)[:, None, None]
    flat_idx = (idx.astype(jnp.int32) + off).reshape(-1)
    out = _make_sc_gather(Bb * V, Dp, flat_idx.shape[0])(table.reshape(-1, Dp), flat_idx)
    return out.reshape(Bb, N, k, Dp)[..., :D]


def _pad16(xyz):
    return jnp.pad(xyz, ((0, 0), (0, 0), (0, 13)))


def _make_knn_tc(B, Nq, Ns, K, QB, r2, interpret=False):
    grid = (B, Nq // QB)

    def body(q_ref, s_ref, bb_ref, oidx_ref, d_scr):
        q = q_ref[0]
        s = s_ref[0]
        bb = bb_ref[0]
        ab = jax.lax.dot_general(q, s, dimension_numbers=(((1,), (1,)), ((), ())),
                                 preferred_element_type=jnp.float32)
        aa = jnp.sum(q * q, axis=1, keepdims=True)
        d_scr[...] = jnp.maximum(aa + bb - 2.0 * ab, 0.0)
        iota = jax.lax.broadcasted_iota(jnp.int32, (QB, Ns), 1)
        kio = jax.lax.broadcasted_iota(jnp.int32, (QB, K), 1)

        def step(j, carry):
            mprev, am0, acc = carry
            d = d_scr[...]
            t = jnp.where(d > mprev, d, jnp.float32(jnp.inf))
            m = jnp.min(t, axis=1, keepdims=True)
            sel = jnp.where(d == m, iota, Ns)
            am = jnp.min(sel, axis=1, keepdims=True)
            am = jnp.minimum(am, Ns - 1)
            am0 = jnp.where(j == 0, am, am0)
            res = am if r2 is None else jnp.where(m <= r2, am, am0)
            acc = jnp.where(kio == j, res, acc)
            return m, am0, acc

        _, _, acc = jax.lax.fori_loop(
            0, K, step,
            (jnp.full((QB, 1), -jnp.inf, jnp.float32),
             jnp.zeros((QB, 1), jnp.int32), jnp.zeros((QB, K), jnp.int32)))
        oidx_ref[0] = acc

    return pl.pallas_call(
        body,
        grid=grid,
        in_specs=[
            pl.BlockSpec((1, QB, 3), lambda b, i: (b, i, 0)),
            pl.BlockSpec((1, Ns, 3), lambda b, i: (b, 0, 0)),
            pl.BlockSpec((1, 1, Ns), lambda b, i: (b, 0, 0)),
        ],
        out_specs=pl.BlockSpec((1, QB, K), lambda b, i: (b, i, 0)),
        out_shape=jax.ShapeDtypeStruct((B, Nq, K), jnp.int32),
        scratch_shapes=[pltpu.VMEM((QB, Ns), jnp.float32)],
        interpret=interpret,
    )


def _knn_tc(k, support, query, r2=None, qb=256, interpret=False):
    B, Nq, _ = query.shape
    Ns = support.shape[1]
    bb = jnp.sum(support * support, axis=-1)[:, None, :]
    fn = _make_knn_tc(B, Nq, Ns, k, qb, r2, interpret)
    return fn(query, support, bb)


def _sqdist(a, b):
    aa = jnp.sum(a * a, axis=-1)[:, :, None]
    bb = jnp.sum(b * b, axis=-1)[:, None, :]
    ab = jnp.einsum('bnc,bmc->bnm', a, b)
    return jnp.maximum(aa + bb - 2.0 * ab, 0.0)


def _knn(k, support, query):
    d = _sqdist(query, support)
    negd, idx = jax.lax.top_k(-d, k)
    return idx, -negd


def _ball_query(radius, k, support, query):
    idx, d = _knn(k, support, query)
    mask = d <= radius * radius
    return jnp.where(mask, idx, idx[:, :, :1])


def _bn(x, g, b, eps=1e-5):
    axes = tuple(range(x.ndim - 1))
    m = jnp.mean(x, axis=axes, keepdims=True)
    v = jnp.var(x, axis=axes, keepdims=True)
    return (x - m) / jnp.sqrt(v + eps) * g + b


def _ln(x, g, b, eps=1e-5):
    m = jnp.mean(x, axis=-1, keepdims=True)
    v = jnp.var(x, axis=-1, keepdims=True)
    return (x - m) / jnp.sqrt(v + eps) * g + b


def _ff(xyz, feat, idx, Wg, bg, Wf, bf):
    g = _sc_gather(jnp.concatenate([_pad16(xyz), feat], axis=-1), idx)
    nx = g[..., :3]
    nf = g[..., 16:]
    rel = nx - xyz[:, :, None, :]
    geo = jax.nn.relu(rel @ Wg + bg)
    h = jax.nn.relu(jnp.concatenate([nf, geo], axis=-1) @ Wf + bf)
    return jnp.max(h, axis=2)


def _agg(xyz1, xyz2, f1, f2, idx, Wq, Wk, Wv, Wgeo, Wo, bo, H):
    B, N2, k = idx.shape
    C = Wq.shape[1]
    dh = C // H
    g = _sc_gather(jnp.concatenate([_pad16(xyz1), f1], axis=-1), idx)
    nx = g[..., :3]
    nf = g[..., 16:]
    rel = nx - xyz2[:, :, None, :]
    kv = nf + rel @ Wgeo
    q = (f2 @ Wq).reshape(B, N2, H, dh)
    kk = (kv @ Wk).reshape(B, N2, k, H, dh)
    v = (kv @ Wv).reshape(B, N2, k, H, dh)
    att = jax.nn.softmax(jnp.einsum('bnhd,bnkhd->bnhk', q, kk) / np.sqrt(dh), axis=-1)
    o = jnp.einsum('bnhk,bnkhd->bnhd', att, v).reshape(B, N2, C)
    return jax.nn.relu(o @ Wo + bo)


def _glob(f, p):
    B, N, C = f.shape
    H = 8
    dh = C // H
    h = _ln(f, p['ln1_g'], p['ln1_b'])
    q = (h @ p['Wqg']).reshape(B, N, H, dh)
    k = (h @ p['Wkg']).reshape(B, N, H, dh)
    v = (h @ p['Wvg']).reshape(B, N, H, dh)
    att = jax.nn.softmax(jnp.einsum('bnhd,bmhd->bhnm', q, k) / np.sqrt(dh), axis=-1)
    o = jnp.einsum('bhnm,bmhd->bnhd', att, v).reshape(B, N, C) @ p['Wog']
    f = f + o
    h = _ln(f, p['ln2_g'], p['ln2_b'])
    f = f + jax.nn.relu(h @ p['Wff1'] + p['bff1']) @ p['Wff2'] + p['bff2']
    return f


def _up(fc, fskip, xyz_c, xyz_f, W1, b1, W2, b2):
    d = _sqdist(xyz_f, xyz_c)
    negd, idx3 = jax.lax.top_k(-d, 3)
    d3 = jnp.maximum(-negd, 1e-10)
    w = 1.0 / d3
    w = w / jnp.sum(w, axis=-1, keepdims=True)
    f3 = _sc_gather(fc, idx3)
    fi = jnp.sum(w[..., None] * f3, axis=2)
    h = jnp.concatenate([fi, fskip], axis=-1)
    h = jax.nn.relu(h @ W1 + b1)
    return jax.nn.relu(h @ W2 + b2)


def _head_body(feat_ref, We1_ref, be1_ref, g1_ref, b1_ref,
               We2_ref, be2_ref, g2_ref, b2_ref,
               We3_ref, be3_ref, g3_ref, b3_ref, out_ref):
    feat = feat_ref[...]

    def bn(x, g, b, eps=1e-5):
        m = jnp.mean(x, axis=(0, 1), keepdims=True)
        v = jnp.mean((x - m) * (x - m), axis=(0, 1), keepdims=True)
        return (x - m) / jnp.sqrt(v + eps) * g + b

    h = jax.nn.relu(bn(feat @ We1_ref[...] + be1_ref[...], g1_ref[...], b1_ref[...]))
    h = jax.nn.relu(bn(h @ We2_ref[...] + be2_ref[...], g2_ref[...], b2_ref[...]))
    h = bn(h @ We3_ref[...] + be3_ref[...], g3_ref[...], b3_ref[...])
    out_ref[...] = jax.nn.log_softmax(h, axis=-1)


def _head(feat, p):
    B, N, _ = feat.shape
    args = (feat, p['We1'], p['be1'], p['g1'], p['b1'],
            p['We2'], p['be2'], p['g2'], p['b2'],
            p['We3'], p['be3'], p['g3'], p['b3'])
    return pl.pallas_call(
        _head_body,
        out_shape=jax.ShapeDtypeStruct((B, N, 13), jnp.float32),
    )(*args)


def kernel(points, params):
    p = params
    xyz = points[..., 0:3]
    xyz2 = xyz[:, ::4]
    xyz3 = xyz2[:, ::4]
    idxs = {
        's0': _knn_tc(32, xyz, xyz, r2=0.01),
        'a0': _knn_tc(16, xyz, xyz2),
        's1': _knn_tc(32, xyz2, xyz2, r2=0.04),
        'a1': _knn_tc(16, xyz2, xyz3),
    }
    feat = jax.nn.relu(_bn(xyz @ p['W_emb'] + p['b_emb'], p['g_emb'], p['be_emb']))
    f1 = _ff(xyz, feat, idxs['s0'], p['Wg0'], p['bg0'], p['Wf0'], p['bf0'])
    enc0 = f1
    f1s = f1[:, ::4]
    feat = _agg(xyz, xyz2, f1, f1s, idxs['a0'], p['Wq0'], p['Wk0'], p['Wv0'], p['Wgeo0'], p['Wo0'], p['bo0'], 4)
    enc1 = feat
    f1b = _ff(xyz2, feat, idxs['s1'], p['Wg1'], p['bg1'], p['Wf1'], p['bf1'])
    f1bs = f1b[:, ::4]
    feat = _agg(xyz2, xyz3, f1b, f1bs, idxs['a1'], p['Wq1'], p['Wk1'], p['Wv1'], p['Wgeo1'], p['Wo1'], p['bo1'], 4)
    feat = _glob(feat, p)
    feat = _up(feat, enc1, xyz3, xyz2, p['Wu0a'], p['bu0a'], p['Wu0b'], p['bu0b'])
    feat = _up(feat, enc0, xyz2, xyz, p['Wu1a'], p['bu1a'], p['Wu1b'], p['bu1b'])
    return _head(feat, p)


# speedup vs baseline: 6.3664x; 1.0226x over previous
import functools

import jax
import jax.numpy as jnp
import numpy as np
from jax import lax
from jax.experimental import pallas as pl
from jax.experimental.pallas import tpu as pltpu
from jax.experimental.pallas import tpu_sc as plsc

_NC = 2
_NS = 16
_NW = _NC * _NS


@functools.cache
def _make_sc_gather(V, D, B):
    assert D % 16 == 0 and B % (8 * _NW) == 0
    b_per_w = B // _NW
    CH = min(128, b_per_w)
    n_ch = b_per_w // CH
    assert b_per_w % CH == 0
    mesh = plsc.VectorSubcoreMesh(core_axis_name="c", subcore_axis_name="s")

    @functools.partial(
        pl.kernel, mesh=mesh,
        out_type=jax.ShapeDtypeStruct((B, D), jnp.float32),
        scratch_types=[
            pltpu.VMEM((CH,), jnp.int32),
            pltpu.VMEM((CH, D), jnp.float32),
            pltpu.SemaphoreType.DMA,
        ],
    )
    def k(table_hbm, idx_hbm, out_hbm, idx_v, rows_v, sem):
        wid = lax.axis_index("s") * _NC + lax.axis_index("c")
        base = wid * b_per_w

        def chunk(i, carry):
            off = base + i * CH
            pltpu.sync_copy(idx_hbm.at[pl.ds(off, CH)], idx_v)
            pltpu.async_copy(table_hbm.at[idx_v], rows_v, sem).wait()
            pltpu.sync_copy(rows_v, out_hbm.at[pl.ds(off, CH)])
            return carry

        lax.fori_loop(0, n_ch, chunk, 0)

    return k


def _sc_gather(table, idx):
    Bb, V, D = table.shape
    _, N, k = idx.shape
    Dp = ((D + 127) // 128) * 128
    if Dp != D:
        table = jnp.pad(table, ((0, 0), (0, 0), (0, Dp - D)))
    off = (jnp.arange(Bb, dtype=jnp.int32) * V)[:, None, None]
    flat_idx = (idx.astype(jnp.int32) + off).reshape(-1)
    out = _make_sc_gather(Bb * V, Dp, flat_idx.shape[0])(table.reshape(-1, Dp), flat_idx)
    return out.reshape(Bb, N, k, Dp)[..., :D]


def _sc_gather_flat(table, idx):
    Bb, V, D = table.shape
    Dp = ((D + 127) // 128) * 128
    if Dp != D:
        table = jnp.pad(table, ((0, 0), (0, 0), (0, Dp - D)))
    off = (jnp.arange(Bb, dtype=jnp.int32) * V)[:, None, None]
    flat_idx = (idx.astype(jnp.int32) + off).reshape(-1)
    return _make_sc_gather(Bb * V, Dp, flat_idx.shape[0])(table.reshape(-1, Dp), flat_idx)


def _pad16(xyz):
    return jnp.pad(xyz, ((0, 0), (0, 0), (0, 13)))


def _make_ff_tc(Ntot, k, C, Cout, RB, interpret=False):
    grid = (Ntot // RB,)

    def body(g_ref, x_ref, wg_ref, bg_ref, wfa_ref, wfb_ref, bf_ref, out_ref):
        G = g_ref[...]
        nx = G[:, :3]
        nf = G[:, 16:16 + C]
        x = x_ref[...]
        nxg = nx @ wg_ref[...]
        xg = x @ wg_ref[...]
        geo = jax.nn.relu(
            nxg.reshape(RB, k, 16) - xg[:, None, :] + bg_ref[...])
        h = jax.nn.relu(
            nf @ wfa_ref[...]
            + geo.reshape(RB * k, 16) @ wfb_ref[...] + bf_ref[...])
        out_ref[...] = jnp.max(h.reshape(RB, k, Cout), axis=1)

    return pl.pallas_call(
        body,
        grid=grid,
        in_specs=[
            pl.BlockSpec((RB * k, 128 if C <= 112 else 256), lambda i: (i, 0)),
            pl.BlockSpec((RB, 3), lambda i: (i, 0)),
            pl.BlockSpec((3, 16), lambda i: (0, 0)),
            pl.BlockSpec((1, 16), lambda i: (0, 0)),
            pl.BlockSpec((C, Cout), lambda i: (0, 0)),
            pl.BlockSpec((16, Cout), lambda i: (0, 0)),
            pl.BlockSpec((1, Cout), lambda i: (0, 0)),
        ],
        out_specs=pl.BlockSpec((RB, Cout), lambda i: (i, 0)),
        out_shape=jax.ShapeDtypeStruct((Ntot, Cout), jnp.float32),
        interpret=interpret,
    )


def _ff_fused(xyz, feat, idx, Wg, bg, Wf, bf, interpret=False):
    B, N, _ = xyz.shape
    k = idx.shape[-1]
    C = feat.shape[-1]
    Cout = Wf.shape[1]
    gflat = _sc_gather_flat(jnp.concatenate([_pad16(xyz), feat], axis=-1), idx)
    fn = _make_ff_tc(B * N, k, C, Cout, 256, interpret)
    out = fn(gflat, xyz.reshape(B * N, 3), Wg, bg[None, :],
             Wf[:C], Wf[C:], bf[None, :])
    return out.reshape(B, N, Cout)


def _make_knn_tc(B, Nq, Ns, K, QB, r2, interpret=False):
    grid = (B, Nq // QB)

    def body(q_ref, s_ref, bb_ref, oidx_ref, d_scr):
        q = q_ref[0]
        s = s_ref[0]
        bb = bb_ref[0]
        ab = jax.lax.dot_general(q, s, dimension_numbers=(((1,), (1,)), ((), ())),
                                 preferred_element_type=jnp.float32)
        aa = jnp.sum(q * q, axis=1, keepdims=True)
        d_scr[...] = jnp.maximum(aa + bb - 2.0 * ab, 0.0)
        iota = jax.lax.broadcasted_iota(jnp.int32, (QB, Ns), 1)
        kio = jax.lax.broadcasted_iota(jnp.int32, (QB, K), 1)

        def step(j, carry):
            am0, acc = carry
            d = d_scr[...]
            m = jnp.min(d, axis=1, keepdims=True)
            sel = jnp.where(d <= m, iota, Ns)
            am = jnp.min(sel, axis=1, keepdims=True)
            am0 = jnp.where(j == 0, am, am0)
            res = am if r2 is None else jnp.where(m <= r2, am, am0)
            acc = jnp.where(kio == j, res, acc)
            d_scr[...] = jnp.where(sel == am, jnp.float32(jnp.inf), d)
            return am0, acc

        _, acc = jax.lax.fori_loop(
            0, K, step,
            (jnp.zeros((QB, 1), jnp.int32), jnp.zeros((QB, K), jnp.int32)))
        oidx_ref[0] = acc

    return pl.pallas_call(
        body,
        grid=grid,
        in_specs=[
            pl.BlockSpec((1, QB, 3), lambda b, i: (b, i, 0)),
            pl.BlockSpec((1, Ns, 3), lambda b, i: (b, 0, 0)),
            pl.BlockSpec((1, 1, Ns), lambda b, i: (b, 0, 0)),
        ],
        out_specs=pl.BlockSpec((1, QB, K), lambda b, i: (b, i, 0)),
        out_shape=jax.ShapeDtypeStruct((B, Nq, K), jnp.int32),
        scratch_shapes=[pltpu.VMEM((QB, Ns), jnp.float32)],
        interpret=interpret,
    )


def _knn_tc(k, support, query, r2=None, qb=256, interpret=False):
    B, Nq, _ = query.shape
    Ns = support.shape[1]
    bb = jnp.sum(support * support, axis=-1)[:, None, :]
    fn = _make_knn_tc(B, Nq, Ns, k, qb, r2, interpret)
    return fn(query, support, bb)


def _sqdist(a, b):
    aa = jnp.sum(a * a, axis=-1)[:, :, None]
    bb = jnp.sum(b * b, axis=-1)[:, None, :]
    ab = jnp.einsum('bnc,bmc->bnm', a, b)
    return jnp.maximum(aa + bb - 2.0 * ab, 0.0)


def _knn(k, support, query):
    d = _sqdist(query, support)
    negd, idx = jax.lax.top_k(-d, k)
    return idx, -negd


def _ball_query(radius, k, support, query):
    idx, d = _knn(k, support, query)
    mask = d <= radius * radius
    return jnp.where(mask, idx, idx[:, :, :1])


def _bn(x, g, b, eps=1e-5):
    axes = tuple(range(x.ndim - 1))
    m = jnp.mean(x, axis=axes, keepdims=True)
    v = jnp.var(x, axis=axes, keepdims=True)
    return (x - m) / jnp.sqrt(v + eps) * g + b


def _ln(x, g, b, eps=1e-5):
    m = jnp.mean(x, axis=-1, keepdims=True)
    v = jnp.var(x, axis=-1, keepdims=True)
    return (x - m) / jnp.sqrt(v + eps) * g + b


def _ff(xyz, feat, idx, Wg, bg, Wf, bf):
    g = _sc_gather(jnp.concatenate([_pad16(xyz), feat], axis=-1), idx)
    nx = g[..., :3]
    nf = g[..., 16:]
    rel = nx - xyz[:, :, None, :]
    geo = jax.nn.relu(rel @ Wg + bg)
    h = jax.nn.relu(jnp.concatenate([nf, geo], axis=-1) @ Wf + bf)
    return jnp.max(h, axis=2)


def _agg(xyz1, xyz2, f1, f2, idx, Wq, Wk, Wv, Wgeo, Wo, bo, H):
    B, N2, k = idx.shape
    C = Wq.shape[1]
    dh = C // H
    g = _sc_gather(jnp.concatenate([_pad16(xyz1), f1], axis=-1), idx)
    nx = g[..., :3]
    nf = g[..., 16:]
    rel = nx - xyz2[:, :, None, :]
    kv = nf + rel @ Wgeo
    q = (f2 @ Wq).reshape(B, N2, H, dh)
    kk = (kv @ Wk).reshape(B, N2, k, H, dh)
    v = (kv @ Wv).reshape(B, N2, k, H, dh)
    att = jax.nn.softmax(jnp.einsum('bnhd,bnkhd->bnhk', q, kk) / np.sqrt(dh), axis=-1)
    o = jnp.einsum('bnhk,bnkhd->bnhd', att, v).reshape(B, N2, C)
    return jax.nn.relu(o @ Wo + bo)


def _glob(f, p):
    B, N, C = f.shape
    H = 8
    dh = C // H
    h = _ln(f, p['ln1_g'], p['ln1_b'])
    q = (h @ p['Wqg']).reshape(B, N, H, dh)
    k = (h @ p['Wkg']).reshape(B, N, H, dh)
    v = (h @ p['Wvg']).reshape(B, N, H, dh)
    att = jax.nn.softmax(jnp.einsum('bnhd,bmhd->bhnm', q, k) / np.sqrt(dh), axis=-1)
    o = jnp.einsum('bhnm,bmhd->bnhd', att, v).reshape(B, N, C) @ p['Wog']
    f = f + o
    h = _ln(f, p['ln2_g'], p['ln2_b'])
    f = f + jax.nn.relu(h @ p['Wff1'] + p['bff1']) @ p['Wff2'] + p['bff2']
    return f


def _up(fc, fskip, xyz_c, xyz_f, W1, b1, W2, b2):
    d = _sqdist(xyz_f, xyz_c)
    negd, idx3 = jax.lax.top_k(-d, 3)
    d3 = jnp.maximum(-negd, 1e-10)
    w = 1.0 / d3
    w = w / jnp.sum(w, axis=-1, keepdims=True)
    f3 = _sc_gather(fc, idx3)
    fi = jnp.sum(w[..., None] * f3, axis=2)
    h = jnp.concatenate([fi, fskip], axis=-1)
    h = jax.nn.relu(h @ W1 + b1)
    return jax.nn.relu(h @ W2 + b2)


def _head_body(feat_ref, We1_ref, be1_ref, g1_ref, b1_ref,
               We2_ref, be2_ref, g2_ref, b2_ref,
               We3_ref, be3_ref, g3_ref, b3_ref, out_ref):
    feat = feat_ref[...]

    def bn(x, g, b, eps=1e-5):
        m = jnp.mean(x, axis=(0, 1), keepdims=True)
        v = jnp.mean((x - m) * (x - m), axis=(0, 1), keepdims=True)
        return (x - m) / jnp.sqrt(v + eps) * g + b

    h = jax.nn.relu(bn(feat @ We1_ref[...] + be1_ref[...], g1_ref[...], b1_ref[...]))
    h = jax.nn.relu(bn(h @ We2_ref[...] + be2_ref[...], g2_ref[...], b2_ref[...]))
    h = bn(h @ We3_ref[...] + be3_ref[...], g3_ref[...], b3_ref[...])
    out_ref[...] = jax.nn.log_softmax(h, axis=-1)


def _head(feat, p):
    B, N, _ = feat.shape
    args = (feat, p['We1'], p['be1'], p['g1'], p['b1'],
            p['We2'], p['be2'], p['g2'], p['b2'],
            p['We3'], p['be3'], p['g3'], p['b3'])
    return pl.pallas_call(
        _head_body,
        out_shape=jax.ShapeDtypeStruct((B, N, 13), jnp.float32),
    )(*args)


def kernel(points, params):
    p = params
    xyz = points[..., 0:3]
    xyz2 = xyz[:, ::4]
    xyz3 = xyz2[:, ::4]
    idxs = {
        's0': _knn_tc(32, xyz, xyz, r2=0.01),
        'a0': _knn_tc(16, xyz, xyz2),
        's1': _knn_tc(32, xyz2, xyz2, r2=0.04),
        'a1': _knn_tc(16, xyz2, xyz3),
    }
    feat = jax.nn.relu(_bn(xyz @ p['W_emb'] + p['b_emb'], p['g_emb'], p['be_emb']))
    f1 = _ff_fused(xyz, feat, idxs['s0'], p['Wg0'], p['bg0'], p['Wf0'], p['bf0'])
    enc0 = f1
    f1s = f1[:, ::4]
    feat = _agg(xyz, xyz2, f1, f1s, idxs['a0'], p['Wq0'], p['Wk0'], p['Wv0'], p['Wgeo0'], p['Wo0'], p['bo0'], 4)
    enc1 = feat
    f1b = _ff_fused(xyz2, feat, idxs['s1'], p['Wg1'], p['bg1'], p['Wf1'], p['bf1'])
    f1bs = f1b[:, ::4]
    feat = _agg(xyz2, xyz3, f1b, f1bs, idxs['a1'], p['Wq1'], p['Wk1'], p['Wv1'], p['Wgeo1'], p['Wo1'], p['bo1'], 4)
    feat = _glob(feat, p)
    feat = _up(feat, enc1, xyz3, xyz2, p['Wu0a'], p['bu0a'], p['Wu0b'], p['bu0b'])
    feat = _up(feat, enc0, xyz2, xyz, p['Wu1a'], p['bu1a'], p['Wu1b'], p['bu1b'])
    return _head(feat, p)


# speedup vs baseline: 6.3715x; 1.0008x over previous
import functools

import jax
import jax.numpy as jnp
import numpy as np
from jax import lax
from jax.experimental import pallas as pl
from jax.experimental.pallas import tpu as pltpu
from jax.experimental.pallas import tpu_sc as plsc

_NC = 2
_NS = 16
_NW = _NC * _NS


@functools.cache
def _make_sc_gather(V, D, B):
    assert D % 16 == 0 and B % (8 * _NW) == 0
    b_per_w = B // _NW
    CH = min(128, b_per_w)
    n_ch = b_per_w // CH
    assert b_per_w % CH == 0
    mesh = plsc.VectorSubcoreMesh(core_axis_name="c", subcore_axis_name="s")

    @functools.partial(
        pl.kernel, mesh=mesh,
        out_type=jax.ShapeDtypeStruct((B, D), jnp.float32),
        scratch_types=[
            pltpu.VMEM((CH,), jnp.int32),
            pltpu.VMEM((CH, D), jnp.float32),
            pltpu.SemaphoreType.DMA,
        ],
    )
    def k(table_hbm, idx_hbm, out_hbm, idx_v, rows_v, sem):
        wid = lax.axis_index("s") * _NC + lax.axis_index("c")
        base = wid * b_per_w

        def chunk(i, carry):
            off = base + i * CH
            pltpu.sync_copy(idx_hbm.at[pl.ds(off, CH)], idx_v)
            pltpu.async_copy(table_hbm.at[idx_v], rows_v, sem).wait()
            pltpu.sync_copy(rows_v, out_hbm.at[pl.ds(off, CH)])
            return carry

        lax.fori_loop(0, n_ch, chunk, 0)

    return k


def _sc_gather(table, idx):
    Bb, V, D = table.shape
    _, N, k = idx.shape
    Dp = ((D + 127) // 128) * 128
    if Dp != D:
        table = jnp.pad(table, ((0, 0), (0, 0), (0, Dp - D)))
    off = (jnp.arange(Bb, dtype=jnp.int32) * V)[:, None, None]
    flat_idx = (idx.astype(jnp.int32) + off).reshape(-1)
    out = _make_sc_gather(Bb * V, Dp, flat_idx.shape[0])(table.reshape(-1, Dp), flat_idx)
    return out.reshape(Bb, N, k, Dp)[..., :D]


def _sc_gather_flat(table, idx):
    Bb, V, D = table.shape
    Dp = ((D + 127) // 128) * 128
    if Dp != D:
        table = jnp.pad(table, ((0, 0), (0, 0), (0, Dp - D)))
    off = (jnp.arange(Bb, dtype=jnp.int32) * V)[:, None, None]
    flat_idx = (idx.astype(jnp.int32) + off).reshape(-1)
    return _make_sc_gather(Bb * V, Dp, flat_idx.shape[0])(table.reshape(-1, Dp), flat_idx)


def _pad16(xyz):
    return jnp.pad(xyz, ((0, 0), (0, 0), (0, 13)))


def _make_ff_tc(Ntot, k, C, Cout, RB, interpret=False):
    grid = (Ntot // RB,)

    def body(g_ref, x_ref, wg_ref, bg_ref, wfa_ref, wfb_ref, bf_ref, out_ref):
        G = g_ref[...]
        nx = G[:, :3]
        nf = G[:, 16:16 + C]
        x = x_ref[...]
        rel = (nx.reshape(RB, k, 3) - x[:, None, :]).reshape(RB * k, 3)
        geo = jax.nn.relu(rel @ wg_ref[...] + bg_ref[...])
        h = jax.nn.relu(
            nf @ wfa_ref[...] + geo @ wfb_ref[...] + bf_ref[...])
        out_ref[...] = jnp.max(h.reshape(RB, k, Cout), axis=1)

    return pl.pallas_call(
        body,
        grid=grid,
        in_specs=[
            pl.BlockSpec((RB * k, 128 if C <= 112 else 256), lambda i: (i, 0)),
            pl.BlockSpec((RB, 3), lambda i: (i, 0)),
            pl.BlockSpec((3, 16), lambda i: (0, 0)),
            pl.BlockSpec((1, 16), lambda i: (0, 0)),
            pl.BlockSpec((C, Cout), lambda i: (0, 0)),
            pl.BlockSpec((16, Cout), lambda i: (0, 0)),
            pl.BlockSpec((1, Cout), lambda i: (0, 0)),
        ],
        out_specs=pl.BlockSpec((RB, Cout), lambda i: (i, 0)),
        out_shape=jax.ShapeDtypeStruct((Ntot, Cout), jnp.float32),
        interpret=interpret,
    )


def _ff_fused(xyz, feat, idx, Wg, bg, Wf, bf, interpret=False):
    B, N, _ = xyz.shape
    k = idx.shape[-1]
    C = feat.shape[-1]
    Cout = Wf.shape[1]
    gflat = _sc_gather_flat(jnp.concatenate([_pad16(xyz), feat], axis=-1), idx)
    fn = _make_ff_tc(B * N, k, C, Cout, 256, interpret)
    out = fn(gflat, xyz.reshape(B * N, 3), Wg, bg[None, :],
             Wf[:C], Wf[C:], bf[None, :])
    return out.reshape(B, N, Cout)


def _make_knn_tc(B, Nq, Ns, K, QB, r2, interpret=False):
    grid = (B, Nq // QB)

    def body(q_ref, s_ref, bb_ref, oidx_ref, d_scr):
        q = q_ref[0]
        s = s_ref[0]
        bb = bb_ref[0]
        ab = jax.lax.dot_general(q, s, dimension_numbers=(((1,), (1,)), ((), ())),
                                 preferred_element_type=jnp.float32)
        aa = jnp.sum(q * q, axis=1, keepdims=True)
        d_scr[...] = jnp.maximum(aa + bb - 2.0 * ab, 0.0)
        iota = jax.lax.broadcasted_iota(jnp.int32, (QB, Ns), 1)
        kio = jax.lax.broadcasted_iota(jnp.int32, (QB, K), 1)

        def step(j, carry):
            am0, acc = carry
            d = d_scr[...]
            m = jnp.min(d, axis=1, keepdims=True)
            sel = jnp.where(d <= m, iota, Ns)
            am = jnp.min(sel, axis=1, keepdims=True)
            am0 = jnp.where(j == 0, am, am0)
            res = am if r2 is None else jnp.where(m <= r2, am, am0)
            acc = jnp.where(kio == j, res, acc)
            d_scr[...] = jnp.where(sel == am, jnp.float32(jnp.inf), d)
            return am0, acc

        _, acc = jax.lax.fori_loop(
            0, K, step,
            (jnp.zeros((QB, 1), jnp.int32), jnp.zeros((QB, K), jnp.int32)))
        oidx_ref[0] = acc

    return pl.pallas_call(
        body,
        grid=grid,
        in_specs=[
            pl.BlockSpec((1, QB, 3), lambda b, i: (b, i, 0)),
            pl.BlockSpec((1, Ns, 3), lambda b, i: (b, 0, 0)),
            pl.BlockSpec((1, 1, Ns), lambda b, i: (b, 0, 0)),
        ],
        out_specs=pl.BlockSpec((1, QB, K), lambda b, i: (b, i, 0)),
        out_shape=jax.ShapeDtypeStruct((B, Nq, K), jnp.int32),
        scratch_shapes=[pltpu.VMEM((QB, Ns), jnp.float32)],
        interpret=interpret,
    )


def _knn_tc(k, support, query, r2=None, qb=256, interpret=False):
    B, Nq, _ = query.shape
    Ns = support.shape[1]
    bb = jnp.sum(support * support, axis=-1)[:, None, :]
    fn = _make_knn_tc(B, Nq, Ns, k, qb, r2, interpret)
    return fn(query, support, bb)


def _sqdist(a, b):
    aa = jnp.sum(a * a, axis=-1)[:, :, None]
    bb = jnp.sum(b * b, axis=-1)[:, None, :]
    ab = jnp.einsum('bnc,bmc->bnm', a, b)
    return jnp.maximum(aa + bb - 2.0 * ab, 0.0)


def _knn(k, support, query):
    d = _sqdist(query, support)
    negd, idx = jax.lax.top_k(-d, k)
    return idx, -negd


def _ball_query(radius, k, support, query):
    idx, d = _knn(k, support, query)
    mask = d <= radius * radius
    return jnp.where(mask, idx, idx[:, :, :1])


def _bn(x, g, b, eps=1e-5):
    axes = tuple(range(x.ndim - 1))
    m = jnp.mean(x, axis=axes, keepdims=True)
    v = jnp.var(x, axis=axes, keepdims=True)
    return (x - m) / jnp.sqrt(v + eps) * g + b


def _ln(x, g, b, eps=1e-5):
    m = jnp.mean(x, axis=-1, keepdims=True)
    v = jnp.var(x, axis=-1, keepdims=True)
    return (x - m) / jnp.sqrt(v + eps) * g + b


def _ff(xyz, feat, idx, Wg, bg, Wf, bf):
    g = _sc_gather(jnp.concatenate([_pad16(xyz), feat], axis=-1), idx)
    nx = g[..., :3]
    nf = g[..., 16:]
    rel = nx - xyz[:, :, None, :]
    geo = jax.nn.relu(rel @ Wg + bg)
    h = jax.nn.relu(jnp.concatenate([nf, geo], axis=-1) @ Wf + bf)
    return jnp.max(h, axis=2)


def _agg(xyz1, xyz2, f1, f2, idx, Wq, Wk, Wv, Wgeo, Wo, bo, H):
    B, N2, k = idx.shape
    C = Wq.shape[1]
    dh = C // H
    g = _sc_gather(jnp.concatenate([_pad16(xyz1), f1], axis=-1), idx)
    nx = g[..., :3]
    nf = g[..., 16:]
    rel = nx - xyz2[:, :, None, :]
    kv = nf + rel @ Wgeo
    q = (f2 @ Wq).reshape(B, N2, H, dh)
    kk = (kv @ Wk).reshape(B, N2, k, H, dh)
    v = (kv @ Wv).reshape(B, N2, k, H, dh)
    att = jax.nn.softmax(jnp.einsum('bnhd,bnkhd->bnhk', q, kk) / np.sqrt(dh), axis=-1)
    o = jnp.einsum('bnhk,bnkhd->bnhd', att, v).reshape(B, N2, C)
    return jax.nn.relu(o @ Wo + bo)


def _glob(f, p):
    B, N, C = f.shape
    H = 8
    dh = C // H
    h = _ln(f, p['ln1_g'], p['ln1_b'])
    q = (h @ p['Wqg']).reshape(B, N, H, dh)
    k = (h @ p['Wkg']).reshape(B, N, H, dh)
    v = (h @ p['Wvg']).reshape(B, N, H, dh)
    att = jax.nn.softmax(jnp.einsum('bnhd,bmhd->bhnm', q, k) / np.sqrt(dh), axis=-1)
    o = jnp.einsum('bhnm,bmhd->bnhd', att, v).reshape(B, N, C) @ p['Wog']
    f = f + o
    h = _ln(f, p['ln2_g'], p['ln2_b'])
    f = f + jax.nn.relu(h @ p['Wff1'] + p['bff1']) @ p['Wff2'] + p['bff2']
    return f


def _up(fc, fskip, xyz_c, xyz_f, W1, b1, W2, b2):
    d = _sqdist(xyz_f, xyz_c)
    negd, idx3 = jax.lax.top_k(-d, 3)
    d3 = jnp.maximum(-negd, 1e-10)
    w = 1.0 / d3
    w = w / jnp.sum(w, axis=-1, keepdims=True)
    f3 = _sc_gather(fc, idx3)
    fi = jnp.sum(w[..., None] * f3, axis=2)
    h = jnp.concatenate([fi, fskip], axis=-1)
    h = jax.nn.relu(h @ W1 + b1)
    return jax.nn.relu(h @ W2 + b2)


def _head_body(feat_ref, We1_ref, be1_ref, g1_ref, b1_ref,
               We2_ref, be2_ref, g2_ref, b2_ref,
               We3_ref, be3_ref, g3_ref, b3_ref, out_ref):
    feat = feat_ref[...]

    def bn(x, g, b, eps=1e-5):
        m = jnp.mean(x, axis=(0, 1), keepdims=True)
        v = jnp.mean((x - m) * (x - m), axis=(0, 1), keepdims=True)
        return (x - m) / jnp.sqrt(v + eps) * g + b

    h = jax.nn.relu(bn(feat @ We1_ref[...] + be1_ref[...], g1_ref[...], b1_ref[...]))
    h = jax.nn.relu(bn(h @ We2_ref[...] + be2_ref[...], g2_ref[...], b2_ref[...]))
    h = bn(h @ We3_ref[...] + be3_ref[...], g3_ref[...], b3_ref[...])
    out_ref[...] = jax.nn.log_softmax(h, axis=-1)


def _head(feat, p):
    B, N, _ = feat.shape
    args = (feat, p['We1'], p['be1'], p['g1'], p['b1'],
            p['We2'], p['be2'], p['g2'], p['b2'],
            p['We3'], p['be3'], p['g3'], p['b3'])
    return pl.pallas_call(
        _head_body,
        out_shape=jax.ShapeDtypeStruct((B, N, 13), jnp.float32),
    )(*args)


def kernel(points, params):
    p = params
    xyz = points[..., 0:3]
    xyz2 = xyz[:, ::4]
    xyz3 = xyz2[:, ::4]
    idxs = {
        's0': _knn_tc(32, xyz, xyz, r2=0.01),
        'a0': _knn_tc(16, xyz, xyz2),
        's1': _knn_tc(32, xyz2, xyz2, r2=0.04),
        'a1': _knn_tc(16, xyz2, xyz3),
    }
    feat = jax.nn.relu(_bn(xyz @ p['W_emb'] + p['b_emb'], p['g_emb'], p['be_emb']))
    f1 = _ff_fused(xyz, feat, idxs['s0'], p['Wg0'], p['bg0'], p['Wf0'], p['bf0'])
    enc0 = f1
    f1s = f1[:, ::4]
    feat = _agg(xyz, xyz2, f1, f1s, idxs['a0'], p['Wq0'], p['Wk0'], p['Wv0'], p['Wgeo0'], p['Wo0'], p['bo0'], 4)
    enc1 = feat
    f1b = _ff_fused(xyz2, feat, idxs['s1'], p['Wg1'], p['bg1'], p['Wf1'], p['bf1'])
    f1bs = f1b[:, ::4]
    feat = _agg(xyz2, xyz3, f1b, f1bs, idxs['a1'], p['Wq1'], p['Wk1'], p['Wv1'], p['Wgeo1'], p['Wo1'], p['bo1'], 4)
    feat = _glob(feat, p)
    feat = _up(feat, enc1, xyz3, xyz2, p['Wu0a'], p['bu0a'], p['Wu0b'], p['bu0b'])
    feat = _up(feat, enc0, xyz2, xyz, p['Wu1a'], p['bu1a'], p['Wu1b'], p['bu1b'])
    return _head(feat, p)
